# initial kernel scaffold (unmeasured)
import jax
import jax.numpy as jnp
from jax import lax
from jax.experimental import pallas as pl
from jax.experimental.pallas import tpu as pltpu


def kernel(
    x,
):
    def body(*refs):
        pass

    out_shape = jax.ShapeDtypeStruct(..., jnp.float32)
    return pl.pallas_call(body, out_shape=out_shape)(...)


# baseline (device time: 11683 ns/iter reference)
import jax
import jax.numpy as jnp
from jax import lax
from jax.experimental import pallas as pl
from jax.experimental.pallas import tpu as pltpu

N_OUT = 512


def kernel(x):
    _, m, n_tot = x.shape

    def body(x_ref, out_ref, send_buf, recv_buf, send_sem, recv_sem):
        px = lax.axis_index("x")
        py = lax.axis_index("y")
        pz = lax.axis_index("z")
        partner = (1 - px, py, pz)

        barrier = pltpu.get_barrier_semaphore()
        pl.semaphore_signal(
            barrier, inc=1, device_id=partner,
            device_id_type=pl.DeviceIdType.MESH,
        )
        pl.semaphore_wait(barrier, 1)

        send_buf[...] = x_ref[0, :, pl.ds((1 - px) * N_OUT, N_OUT)].astype(
            jnp.bfloat16
        )
        rdma = pltpu.make_async_remote_copy(
            src_ref=send_buf,
            dst_ref=recv_buf,
            send_sem=send_sem,
            recv_sem=recv_sem,
            device_id=partner,
            device_id_type=pl.DeviceIdType.MESH,
        )
        rdma.start()
        my_half = x_ref[0, :, pl.ds(px * N_OUT, N_OUT)].astype(jnp.bfloat16)
        rdma.wait()
        out_ref[...] = my_half + recv_buf[...]

    return pl.pallas_call(
        body,
        out_shape=jax.ShapeDtypeStruct((m, N_OUT), jnp.bfloat16),
        in_specs=[pl.BlockSpec(memory_space=pltpu.VMEM)],
        out_specs=pl.BlockSpec(memory_space=pltpu.VMEM),
        scratch_shapes=[
            pltpu.VMEM((m, N_OUT), jnp.bfloat16),
            pltpu.VMEM((m, N_OUT), jnp.bfloat16),
            pltpu.SemaphoreType.DMA,
            pltpu.SemaphoreType.DMA,
        ],
        compiler_params=pltpu.CompilerParams(collective_id=0),
    )(x)


# device time: 11615 ns/iter; 1.0059x vs baseline; 1.0059x over previous
import jax
import jax.numpy as jnp
from jax import lax
from jax.experimental import pallas as pl
from jax.experimental.pallas import tpu as pltpu

N_OUT = 512
NCHUNK = 4


def kernel(x):
    _, m, n_tot = x.shape
    rows = m // NCHUNK

    def body(x_ref, out_ref, send_buf, recv_buf, send_sems, recv_sems):
        px = lax.axis_index("x")
        py = lax.axis_index("y")
        pz = lax.axis_index("z")
        partner = (1 - px, py, pz)

        barrier = pltpu.get_barrier_semaphore()
        pl.semaphore_signal(
            barrier, inc=1, device_id=partner,
            device_id_type=pl.DeviceIdType.MESH,
        )
        pl.semaphore_wait(barrier, 1)

        def make(i):
            sl = pl.ds(i * rows, rows)
            return pltpu.make_async_remote_copy(
                src_ref=send_buf.at[sl],
                dst_ref=recv_buf.at[sl],
                send_sem=send_sems.at[i],
                recv_sem=recv_sems.at[i],
                device_id=partner,
                device_id_type=pl.DeviceIdType.MESH,
            )

        for i in range(NCHUNK):
            sl = pl.ds(i * rows, rows)
            send_buf[sl] = x_ref[0, sl, pl.ds((1 - px) * N_OUT, N_OUT)].astype(
                jnp.bfloat16
            )
            make(i).start()

        for i in range(NCHUNK):
            make(i).wait_recv()
            sl = pl.ds(i * rows, rows)
            out_ref[sl] = (
                x_ref[0, sl, pl.ds(px * N_OUT, N_OUT)].astype(jnp.bfloat16)
                + recv_buf[sl]
            )
        for i in range(NCHUNK):
            make(i).wait_send()

    return pl.pallas_call(
        body,
        out_shape=jax.ShapeDtypeStruct((m, N_OUT), jnp.bfloat16),
        in_specs=[pl.BlockSpec(memory_space=pltpu.VMEM)],
        out_specs=pl.BlockSpec(memory_space=pltpu.VMEM),
        scratch_shapes=[
            pltpu.VMEM((m, N_OUT), jnp.bfloat16),
            pltpu.VMEM((m, N_OUT), jnp.bfloat16),
            pltpu.SemaphoreType.DMA((NCHUNK,)),
            pltpu.SemaphoreType.DMA((NCHUNK,)),
        ],
        compiler_params=pltpu.CompilerParams(collective_id=0),
    )(x)
